# Initial kernel scaffold; baseline (speedup 1.0000x reference)
import jax
import jax.numpy as jnp
from jax.experimental import pallas as pl


def kernel(x, adj, pairs, W1, W2):
    raise NotImplementedError("write your pallas kernel here")


# speedup vs baseline: 1.2194x; 1.2194x over previous
import functools

import jax
import jax.numpy as jnp
from jax import lax
from jax.experimental import pallas as pl
from jax.experimental.pallas import tpu as pltpu
from jax.experimental.pallas import tpu_sc as plsc

_N = 10000
_F = 128
_H = 256
_O = 64
_P = 200000

_BM = 400

_NW = 32
_PW = 6400
_PPAD = _NW * _PW
_CH = 128
_NCH = _PW // _CH


def _gcn1_body(adj_ref, x_ref, w1t_ref, w2t_ref, g_ref):
    t1 = jnp.dot(adj_ref[...], x_ref[...], preferred_element_type=jnp.float32)
    h = jnp.maximum(
        jnp.dot(t1, w1t_ref[...], preferred_element_type=jnp.float32), 0.0)
    g_ref[...] = jnp.dot(h, w2t_ref[...], preferred_element_type=jnp.float32)


def _gcn2_body(adj_ref, g_ref, h2_ref):
    h2_ref[...] = jnp.dot(adj_ref[...], g_ref[...],
                          preferred_element_type=jnp.float32)


def _gcn1(adj, x, w1t, w2t):
    return pl.pallas_call(
        _gcn1_body,
        grid=(_N // _BM,),
        in_specs=[
            pl.BlockSpec((_BM, _N), lambda i: (i, 0)),
            pl.BlockSpec((_N, _F), lambda i: (0, 0)),
            pl.BlockSpec((_F, _H), lambda i: (0, 0)),
            pl.BlockSpec((_H, _O), lambda i: (0, 0)),
        ],
        out_specs=pl.BlockSpec((_BM, _O), lambda i: (i, 0)),
        out_shape=jax.ShapeDtypeStruct((_N, _O), jnp.float32),
    )(adj, x, w1t, w2t)


def _gcn2(adj, g):
    return pl.pallas_call(
        _gcn2_body,
        grid=(_N // _BM,),
        in_specs=[
            pl.BlockSpec((_BM, _N), lambda i: (i, 0)),
            pl.BlockSpec((_N, _O), lambda i: (0, 0)),
        ],
        out_specs=pl.BlockSpec((_BM, _O), lambda i: (i, 0)),
        out_shape=jax.ShapeDtypeStruct((_N, _O), jnp.float32),
    )(adj, g)


def _decode_body(h2_hbm, src_hbm, dst_hbm, out_hbm,
                 sidx, didx, srows, drows, outv, sem):
    wid = lax.axis_index("s") * 2 + lax.axis_index("c")

    def chunk_body(ch, carry):
        base = wid * _PW + ch * _CH
        pltpu.sync_copy(src_hbm.at[pl.ds(base, _CH)], sidx)
        pltpu.sync_copy(dst_hbm.at[pl.ds(base, _CH)], didx)
        pltpu.async_copy(h2_hbm.at[sidx], srows, sem).wait()
        pltpu.async_copy(h2_hbm.at[didx], drows, sem).wait()

        def grp(g, c):
            rows = lax.iota(jnp.int32, 16) + g * 16
            acc = jnp.zeros((16,), jnp.float32)
            for k in range(_O):
                col = jnp.full((16,), k, jnp.int32)
                sv = plsc.load_gather(srows, [rows, col])
                dv = plsc.load_gather(drows, [rows, col])
                acc = acc + sv * dv
            outv[pl.ds(g * 16, 16)] = acc
            return c

        lax.fori_loop(0, _CH // 16, grp, 0)
        pltpu.sync_copy(outv, out_hbm.at[pl.ds(base, _CH)])
        return carry

    lax.fori_loop(0, _NCH, chunk_body, 0)


@functools.cache
def _get_decode():
    return functools.partial(
        pl.kernel,
        mesh=plsc.VectorSubcoreMesh(core_axis_name="c", subcore_axis_name="s"),
        out_type=jax.ShapeDtypeStruct((_PPAD,), jnp.float32),
        scratch_types=[
            pltpu.VMEM((_CH,), jnp.int32),
            pltpu.VMEM((_CH,), jnp.int32),
            pltpu.VMEM((_CH, _O), jnp.float32),
            pltpu.VMEM((_CH, _O), jnp.float32),
            pltpu.VMEM((_CH,), jnp.float32),
            pltpu.SemaphoreType.DMA,
        ],
        compiler_params=pltpu.CompilerParams(
            needs_layout_passes=False, use_tc_tiling_on_sc=False),
    )(_decode_body)


def kernel(x, adj, pairs, W1, W2):
    g = _gcn1(adj, x, W1.T, W2.T)
    h2 = _gcn2(adj, g)
    p32 = pairs.astype(jnp.int32)
    src = jnp.zeros((_PPAD,), jnp.int32).at[:_P].set(p32[:, 0])
    dst = jnp.zeros((_PPAD,), jnp.int32).at[:_P].set(p32[:, 1])
    out = _get_decode()(h2, src, dst)
    return out[:_P]


# speedup vs baseline: 1.8324x; 1.5027x over previous
import functools

import jax
import jax.numpy as jnp
from jax import lax
from jax.experimental import pallas as pl
from jax.experimental.pallas import tpu as pltpu
from jax.experimental.pallas import tpu_sc as plsc

_N = 10000
_F = 128
_H = 256
_O = 64
_P = 200000

_BM = 400

_NW = 32
_PW = 6400
_PPAD = _NW * _PW
_CH = 128
_NCH = _PW // _CH


def _gcn1_body(adj_ref, x_ref, w1t_ref, w2t_ref, g_ref):
    t1 = jnp.dot(adj_ref[...], x_ref[...], preferred_element_type=jnp.float32)
    h = jnp.maximum(
        jnp.dot(t1, w1t_ref[...], preferred_element_type=jnp.float32), 0.0)
    g_ref[...] = jnp.dot(h, w2t_ref[...], preferred_element_type=jnp.float32)


def _gcn2_body(adj_ref, g_ref, h2_ref):
    h2_ref[...] = jnp.dot(adj_ref[...], g_ref[...],
                          preferred_element_type=jnp.float32)


def _gcn1(adj, x, w1t, w2t):
    return pl.pallas_call(
        _gcn1_body,
        grid=(_N // _BM,),
        in_specs=[
            pl.BlockSpec((_BM, _N), lambda i: (i, 0)),
            pl.BlockSpec((_N, _F), lambda i: (0, 0)),
            pl.BlockSpec((_F, _H), lambda i: (0, 0)),
            pl.BlockSpec((_H, _O), lambda i: (0, 0)),
        ],
        out_specs=pl.BlockSpec((_BM, _O), lambda i: (i, 0)),
        out_shape=jax.ShapeDtypeStruct((_N, _O), jnp.float32),
    )(adj, x, w1t, w2t)


def _gcn2(adj, g):
    return pl.pallas_call(
        _gcn2_body,
        grid=(_N // _BM,),
        in_specs=[
            pl.BlockSpec((_BM, _N), lambda i: (i, 0)),
            pl.BlockSpec((_N, _O), lambda i: (0, 0)),
        ],
        out_specs=pl.BlockSpec((_BM, _O), lambda i: (i, 0)),
        out_shape=jax.ShapeDtypeStruct((_N, _O), jnp.float32),
    )(adj, g)


def _decode_body(h2_hbm, src_hbm, dst_hbm, out_hbm,
                 sidx, didx, outv, sr0, sr1, dr0, dr1,
                 sem_s0, sem_s1, sem_d0, sem_d1):
    wid = lax.axis_index("s") * 2 + lax.axis_index("c")
    pltpu.sync_copy(src_hbm.at[wid], sidx)
    pltpu.sync_copy(dst_hbm.at[wid], didx)

    bufs = ((sr0, dr0, sem_s0, sem_d0), (sr1, dr1, sem_s1, sem_d1))

    def fire(ch, sr, dr, ss, sd):
        pltpu.async_copy(h2_hbm.at[sidx.at[ch]], sr, ss)
        pltpu.async_copy(h2_hbm.at[didx.at[ch]], dr, sd)

    fire(0, *bufs[0])
    fire(1, *bufs[1])

    def body(i, carry):
        for b in range(2):
            sr, dr, ss, sd = bufs[b]
            ch = 2 * i + b
            pltpu.make_async_copy(h2_hbm.at[sidx.at[ch]], sr, ss).wait()
            pltpu.make_async_copy(h2_hbm.at[didx.at[ch]], dr, sd).wait()

            def grp(g, c, sr=sr, dr=dr, ch=ch):
                rows = lax.iota(jnp.int32, 16) + g * 16
                acc = jnp.zeros((16,), jnp.float32)
                for k in range(_O):
                    col = jnp.full((16,), k, jnp.int32)
                    acc = acc + (plsc.load_gather(sr, [rows, col])
                                 * plsc.load_gather(dr, [rows, col]))
                outv[ch, pl.ds(g * 16, 16)] = acc
                return c

            lax.fori_loop(0, _CH // 16, grp, 0)
            nxt = ch + 2

            @pl.when(nxt < _NCH)
            def _():
                fire(nxt, sr, dr, ss, sd)
        return carry

    lax.fori_loop(0, _NCH // 2, body, 0)
    pltpu.sync_copy(outv, out_hbm.at[wid])


@functools.cache
def _get_decode():
    return functools.partial(
        pl.kernel,
        mesh=plsc.VectorSubcoreMesh(core_axis_name="c", subcore_axis_name="s"),
        out_type=jax.ShapeDtypeStruct((_NW, _NCH, _CH), jnp.float32),
        scratch_types=[
            pltpu.VMEM((_NCH, _CH), jnp.int32),
            pltpu.VMEM((_NCH, _CH), jnp.int32),
            pltpu.VMEM((_NCH, _CH), jnp.float32),
            pltpu.VMEM((_CH, _O), jnp.float32),
            pltpu.VMEM((_CH, _O), jnp.float32),
            pltpu.VMEM((_CH, _O), jnp.float32),
            pltpu.VMEM((_CH, _O), jnp.float32),
            pltpu.SemaphoreType.DMA,
            pltpu.SemaphoreType.DMA,
            pltpu.SemaphoreType.DMA,
            pltpu.SemaphoreType.DMA,
        ],
        compiler_params=pltpu.CompilerParams(
            needs_layout_passes=False, use_tc_tiling_on_sc=False),
    )(_decode_body)


def kernel(x, adj, pairs, W1, W2):
    g = _gcn1(adj, x, W1.T, W2.T)
    h2 = _gcn2(adj, g)
    p32 = pairs.astype(jnp.int32)
    src = jnp.zeros((_PPAD,), jnp.int32).at[:_P].set(p32[:, 0])
    dst = jnp.zeros((_PPAD,), jnp.int32).at[:_P].set(p32[:, 1])
    out = _get_decode()(h2, src.reshape(_NW, _NCH, _CH),
                        dst.reshape(_NW, _NCH, _CH))
    return out.reshape(_PPAD)[:_P]
